# baseline (speedup 1.0000x reference)
import functools

import jax
import jax.numpy as jnp
from jax import lax
from jax.experimental import pallas as pl
from jax.experimental.pallas import tpu as pltpu
from jax.experimental.pallas import tpu_sc as plsc

B = 4096
S = 200
D_MODEL = 64
SCALE = float(D_MODEL) ** 0.5
NC = 2
NS = 16
NW = NC * NS
L = 16

BBLK = 128
PAIR = 2
TOK = BBLK * PAIR
NBUF = 2
UNITS = S * (B // TOK)
UNITS_PER_W = UNITS // NW
BTB = B // BBLK


TCW = 8192


@functools.cache
def _build_tc_convert(vocab: int):
    grid = (vocab + TCW - 1) // TCW

    def body(x_ref, o_ref):
        x = x_ref[...]
        eye = jnp.eye(D_MODEL, dtype=jnp.float32)
        xt = jax.lax.dot_general(
            x, eye, (((0,), (0,)), ((), ())),
            preferred_element_type=jnp.float32)
        z = xt.reshape(TCW // 2, 2, D_MODEL)
        o_ref[:, 0:D_MODEL] = z[:, 0, :]
        o_ref[:, D_MODEL:2 * D_MODEL] = z[:, 1, :]

    return pl.pallas_call(
        body,
        grid=(grid,),
        in_specs=[pl.BlockSpec((D_MODEL, TCW), lambda i: (0, i))],
        out_specs=pl.BlockSpec((TCW // 2, 2 * D_MODEL), lambda i: (i, 0)),
        out_shape=jax.ShapeDtypeStruct((vocab // 2, 2 * D_MODEL),
                                       jnp.float32),
    )


@functools.cache
def _build(vocab: int):
    mesh = plsc.VectorSubcoreMesh(core_axis_name="c", subcore_axis_name="s")

    @functools.partial(
        pl.kernel,
        mesh=mesh,
        out_type=jax.ShapeDtypeStruct((S, 8, BTB, 8, BBLK), jnp.float32),
        scratch_types=[
            pltpu.VMEM((NBUF, TOK), jnp.int32),
            pltpu.VMEM((NBUF, TOK, D_MODEL), jnp.float32),
            pltpu.VMEM((NBUF, PAIR, 8, 8, BBLK + 1), jnp.float32),
            pltpu.SemaphoreType.DMA,
            pltpu.SemaphoreType.DMA,
            pltpu.SemaphoreType.DMA,
            pltpu.SemaphoreType.DMA,
        ],
        compiler_params=pltpu.CompilerParams(use_tc_tiling_on_sc=False,
                                             needs_layout_passes=False),
    )
    def emb(tokens_hbm, table_hbm, out_hbm, tv, rows_v, blk_v,
            gsem0, gsem1, ssem0, ssem1):
        gsems = (gsem0, gsem1)
        ssems = (ssem0, ssem1)
        wid = lax.axis_index("s") * NC + lax.axis_index("c")
        u0 = wid * UNITS_PER_W
        iota = lax.iota(jnp.int32, L)
        cvecs = [iota + (c * L) for c in range(TOK // L)]

        def unit_su(u):
            s = u // (BTB // PAIR)
            bt = (u % (BTB // PAIR)) * PAIR
            return s, bt

        def fire_gather(u, p):
            s, bt = unit_su(u)
            pltpu.sync_copy(tokens_hbm.at[s, pl.ds(bt * BBLK, TOK)],
                            tv.at[p])
            pltpu.async_copy(table_hbm.at[tv.at[p]], rows_v.at[p], gsems[p])

        def wait_gather(p):
            pltpu.make_async_copy(table_hbm.at[tv.at[p]], rows_v.at[p],
                                  gsems[p]).wait()

        def fire_store(u, p):
            s, bt = unit_su(u)
            for j in range(PAIR):
                pltpu.async_copy(blk_v.at[p, j, :, :, pl.ds(0, BBLK)],
                                 out_hbm.at[s, :, bt + j, :, :], ssems[p])

        def wait_store(p):
            for j in range(PAIR):
                pltpu.make_async_copy(blk_v.at[p, j, :, :, pl.ds(0, BBLK)],
                                      out_hbm.at[0, :, 0, :, :],
                                      ssems[p]).wait()

        dtc = [(iota + c * L) // 8 for c in range(D_MODEL // L)]
        dsc = [(iota + c * L) % 8 for c in range(D_MODEL // L)]

        def transpose_scale(p):
            @plsc.parallel_loop(0, TOK, 1, unroll=4)
            def t_body(t):
                j = t // BBLK
                bs_vec = jnp.full((L,), t % BBLK, jnp.int32)
                for k in range(D_MODEL // L):
                    vals = rows_v[p, t, pl.ds(k * L, L)] * SCALE
                    plsc.store_scatter(blk_v.at[p, j],
                                       [dtc[k], dsc[k], bs_vec], vals)

        for p in range(NBUF):
            fire_gather(u0 + p, p)

        for p in range(NBUF):
            wait_gather(p)
            transpose_scale(p)
            fire_store(u0 + p, p)
            fire_gather(u0 + NBUF + p, p)

        def group_body(gi, acc):
            for p in range(NBUF):
                k = gi * NBUF + p
                wait_gather(p)
                wait_store(p)
                transpose_scale(p)
                fire_store(u0 + k, p)
                fire_gather(u0 + k + NBUF, p)
            return acc

        lax.fori_loop(1, UNITS_PER_W // NBUF - 1, group_body, 0,
                      unroll=False)

        for p in range(NBUF):
            k = UNITS_PER_W - NBUF + p
            wait_gather(p)
            wait_store(p)
            transpose_scale(p)
            fire_store(u0 + k, p)
        for p in range(NBUF):
            wait_store(p)

    return emb


def kernel(tokens, table):
    vocab, d = table.shape
    tokens_t = tokens.T.astype(jnp.int32)
    lin = _build_tc_convert(vocab)(table.T).reshape(vocab, D_MODEL)
    out5 = _build(vocab)(tokens_t, lin)
    out = out5.transpose(2, 4, 0, 1, 3).reshape(B, S, D_MODEL)
    return out


# speedup vs baseline: 1.0805x; 1.0805x over previous
import functools

import jax
import jax.numpy as jnp
from jax import lax
from jax.experimental import pallas as pl
from jax.experimental.pallas import tpu as pltpu
from jax.experimental.pallas import tpu_sc as plsc

B = 4096
S = 200
D_MODEL = 64
SCALE = float(D_MODEL) ** 0.5
NC = 2
NS = 16
NW = NC * NS
L = 16

BBLK = 128
PAIR = 2
TOK = BBLK * PAIR
NBUF = 2
UNITS = S * (B // TOK)
UNITS_PER_W = UNITS // NW
BTB = B // BBLK


TCW = 8192


@functools.cache
def _build_tc_convert(vocab: int):
    grid = (vocab + TCW - 1) // TCW

    def body(x_ref, o_ref):
        xt = x_ref[...].T
        z = xt.reshape(TCW // 2, 2, D_MODEL)
        o_ref[:, 0:D_MODEL] = z[:, 0, :]
        o_ref[:, D_MODEL:2 * D_MODEL] = z[:, 1, :]

    return pl.pallas_call(
        body,
        grid=(grid,),
        in_specs=[pl.BlockSpec((D_MODEL, TCW), lambda i: (0, i))],
        out_specs=pl.BlockSpec((TCW // 2, 2 * D_MODEL), lambda i: (i, 0)),
        out_shape=jax.ShapeDtypeStruct((vocab // 2, 2 * D_MODEL),
                                       jnp.float32),
    )


@functools.cache
def _build(vocab: int):
    mesh = plsc.VectorSubcoreMesh(core_axis_name="c", subcore_axis_name="s")

    @functools.partial(
        pl.kernel,
        mesh=mesh,
        out_type=jax.ShapeDtypeStruct((S, 8, BTB, 8, BBLK), jnp.float32),
        scratch_types=[
            pltpu.VMEM((NBUF, TOK), jnp.int32),
            pltpu.VMEM((NBUF, TOK, D_MODEL), jnp.float32),
            pltpu.VMEM((NBUF, PAIR, 8, 8, BBLK + 1), jnp.float32),
            pltpu.SemaphoreType.DMA,
            pltpu.SemaphoreType.DMA,
            pltpu.SemaphoreType.DMA,
            pltpu.SemaphoreType.DMA,
        ],
        compiler_params=pltpu.CompilerParams(use_tc_tiling_on_sc=False,
                                             needs_layout_passes=False),
    )
    def emb(tokens_hbm, table_hbm, out_hbm, tv, rows_v, blk_v,
            gsem0, gsem1, ssem0, ssem1):
        gsems = (gsem0, gsem1)
        ssems = (ssem0, ssem1)
        wid = lax.axis_index("s") * NC + lax.axis_index("c")
        u0 = wid * UNITS_PER_W
        iota = lax.iota(jnp.int32, L)
        cvecs = [iota + (c * L) for c in range(TOK // L)]

        def unit_su(u):
            s = u // (BTB // PAIR)
            bt = (u % (BTB // PAIR)) * PAIR
            return s, bt

        def fire_gather(u, p):
            s, bt = unit_su(u)
            pltpu.sync_copy(tokens_hbm.at[s, pl.ds(bt * BBLK, TOK)],
                            tv.at[p])
            pltpu.async_copy(table_hbm.at[tv.at[p]], rows_v.at[p], gsems[p])

        def wait_gather(p):
            pltpu.make_async_copy(table_hbm.at[tv.at[p]], rows_v.at[p],
                                  gsems[p]).wait()

        def fire_store(u, p):
            s, bt = unit_su(u)
            for j in range(PAIR):
                pltpu.async_copy(blk_v.at[p, j, :, :, pl.ds(0, BBLK)],
                                 out_hbm.at[s, :, bt + j, :, :], ssems[p])

        def wait_store(p):
            for j in range(PAIR):
                pltpu.make_async_copy(blk_v.at[p, j, :, :, pl.ds(0, BBLK)],
                                      out_hbm.at[0, :, 0, :, :],
                                      ssems[p]).wait()

        dtc = [(iota + c * L) // 8 for c in range(D_MODEL // L)]
        dsc = [(iota + c * L) % 8 for c in range(D_MODEL // L)]

        def transpose_scale(p):
            @plsc.parallel_loop(0, TOK, 1, unroll=4)
            def t_body(t):
                j = t // BBLK
                bs_vec = jnp.full((L,), t % BBLK, jnp.int32)
                for k in range(D_MODEL // L):
                    vals = rows_v[p, t, pl.ds(k * L, L)] * SCALE
                    plsc.store_scatter(blk_v.at[p, j],
                                       [dtc[k], dsc[k], bs_vec], vals)

        for p in range(NBUF):
            fire_gather(u0 + p, p)

        for p in range(NBUF):
            wait_gather(p)
            transpose_scale(p)
            fire_store(u0 + p, p)
            fire_gather(u0 + NBUF + p, p)

        def group_body(gi, acc):
            for p in range(NBUF):
                k = gi * NBUF + p
                wait_gather(p)
                wait_store(p)
                transpose_scale(p)
                fire_store(u0 + k, p)
                fire_gather(u0 + k + NBUF, p)
            return acc

        lax.fori_loop(1, UNITS_PER_W // NBUF - 1, group_body, 0,
                      unroll=False)

        for p in range(NBUF):
            k = UNITS_PER_W - NBUF + p
            wait_gather(p)
            wait_store(p)
            transpose_scale(p)
            fire_store(u0 + k, p)
        for p in range(NBUF):
            wait_store(p)

    return emb


def kernel(tokens, table):
    vocab, d = table.shape
    tokens_t = tokens.T.astype(jnp.int32)
    lin = _build_tc_convert(vocab)(table.T).reshape(vocab, D_MODEL)
    out5 = _build(vocab)(tokens_t, lin)
    out = out5.transpose(2, 4, 0, 1, 3).reshape(B, S, D_MODEL)
    return out


# speedup vs baseline: 1.0957x; 1.0141x over previous
import functools

import jax
import jax.numpy as jnp
from jax import lax
from jax.experimental import pallas as pl
from jax.experimental.pallas import tpu as pltpu
from jax.experimental.pallas import tpu_sc as plsc

B = 4096
S = 200
D_MODEL = 64
SCALE = float(D_MODEL) ** 0.5
NC = 2
NS = 16
NW = NC * NS
L = 16

BBLK = 128
PAIR = 2
TOK = BBLK * PAIR
NBUF = 2
UNITS = S * (B // TOK)
UNITS_PER_W = UNITS // NW
BTB = B // BBLK


TCW = 16384


@functools.cache
def _build_tc_convert(vocab: int):
    grid = (vocab + TCW - 1) // TCW

    def body(x_ref, o_ref):
        xt = x_ref[...].T
        z = xt.reshape(TCW // 2, 2, D_MODEL)
        o_ref[:, 0:D_MODEL] = z[:, 0, :]
        o_ref[:, D_MODEL:2 * D_MODEL] = z[:, 1, :]

    return pl.pallas_call(
        body,
        grid=(grid,),
        in_specs=[pl.BlockSpec((D_MODEL, TCW), lambda i: (0, i))],
        out_specs=pl.BlockSpec((TCW // 2, 2 * D_MODEL), lambda i: (i, 0)),
        out_shape=jax.ShapeDtypeStruct((vocab // 2, 2 * D_MODEL),
                                       jnp.float32),
    )


@functools.cache
def _build(vocab: int):
    mesh = plsc.VectorSubcoreMesh(core_axis_name="c", subcore_axis_name="s")

    @functools.partial(
        pl.kernel,
        mesh=mesh,
        out_type=jax.ShapeDtypeStruct((S, 8, BTB, 8, BBLK), jnp.float32),
        scratch_types=[
            pltpu.VMEM((NBUF, TOK), jnp.int32),
            pltpu.VMEM((NBUF, TOK, D_MODEL), jnp.float32),
            pltpu.VMEM((NBUF, PAIR, 8, 8, BBLK + 1), jnp.float32),
            pltpu.SemaphoreType.DMA,
            pltpu.SemaphoreType.DMA,
            pltpu.SemaphoreType.DMA,
            pltpu.SemaphoreType.DMA,
        ],
        compiler_params=pltpu.CompilerParams(use_tc_tiling_on_sc=False,
                                             needs_layout_passes=False),
    )
    def emb(tokens_hbm, table_hbm, out_hbm, tv, rows_v, blk_v,
            gsem0, gsem1, ssem0, ssem1):
        gsems = (gsem0, gsem1)
        ssems = (ssem0, ssem1)
        wid = lax.axis_index("s") * NC + lax.axis_index("c")
        u0 = wid * UNITS_PER_W
        iota = lax.iota(jnp.int32, L)
        cvecs = [iota + (c * L) for c in range(TOK // L)]

        def unit_su(u):
            s = u // (BTB // PAIR)
            bt = (u % (BTB // PAIR)) * PAIR
            return s, bt

        def fire_gather(u, p):
            s, bt = unit_su(u)
            pltpu.sync_copy(tokens_hbm.at[s, pl.ds(bt * BBLK, TOK)],
                            tv.at[p])
            pltpu.async_copy(table_hbm.at[tv.at[p]], rows_v.at[p], gsems[p])

        def wait_gather(p):
            pltpu.make_async_copy(table_hbm.at[tv.at[p]], rows_v.at[p],
                                  gsems[p]).wait()

        def fire_store(u, p):
            s, bt = unit_su(u)
            for j in range(PAIR):
                pltpu.async_copy(blk_v.at[p, j, :, :, pl.ds(0, BBLK)],
                                 out_hbm.at[s, :, bt + j, :, :], ssems[p])

        def wait_store(p):
            for j in range(PAIR):
                pltpu.make_async_copy(blk_v.at[p, j, :, :, pl.ds(0, BBLK)],
                                      out_hbm.at[0, :, 0, :, :],
                                      ssems[p]).wait()

        dtc = [(iota + c * L) // 8 for c in range(D_MODEL // L)]
        dsc = [(iota + c * L) % 8 for c in range(D_MODEL // L)]

        def transpose_scale(p):
            @plsc.parallel_loop(0, TOK, 1, unroll=4)
            def t_body(t):
                j = t // BBLK
                bs_vec = jnp.full((L,), t % BBLK, jnp.int32)
                for k in range(D_MODEL // L):
                    vals = rows_v[p, t, pl.ds(k * L, L)] * SCALE
                    plsc.store_scatter(blk_v.at[p, j],
                                       [dtc[k], dsc[k], bs_vec], vals)

        for p in range(NBUF):
            fire_gather(u0 + p, p)

        for p in range(NBUF):
            wait_gather(p)
            transpose_scale(p)
            fire_store(u0 + p, p)
            fire_gather(u0 + NBUF + p, p)

        def group_body(gi, acc):
            for p in range(NBUF):
                k = gi * NBUF + p
                wait_gather(p)
                wait_store(p)
                transpose_scale(p)
                fire_store(u0 + k, p)
                fire_gather(u0 + k + NBUF, p)
            return acc

        lax.fori_loop(1, UNITS_PER_W // NBUF - 1, group_body, 0,
                      unroll=False)

        for p in range(NBUF):
            k = UNITS_PER_W - NBUF + p
            wait_gather(p)
            wait_store(p)
            transpose_scale(p)
            fire_store(u0 + k, p)
        for p in range(NBUF):
            wait_store(p)

    return emb


def kernel(tokens, table):
    vocab, d = table.shape
    tokens_t = tokens.T.astype(jnp.int32)
    lin = _build_tc_convert(vocab)(table.T).reshape(vocab, D_MODEL)
    out5 = _build(vocab)(tokens_t, lin)
    out = out5.transpose(2, 4, 0, 1, 3).reshape(B, S, D_MODEL)
    return out
